# baseline (speedup 1.0000x reference)
import jax
import jax.numpy as jnp
from jax.experimental import pallas as pl
from jax.experimental.pallas import tpu as pltpu

_P = 16320
_PP = 16384
_B = 32
_O = 50
_C = 21
_VAR0, _VAR1 = 0.1, 0.2
_THETA = 0.01
_NEGPOS = 3


def _body(targets_ref, targetsT_ref, armloc_ref, armconf_ref, odmloc_ref,
          odmconf_ref, priors_ref, ll_ref, lc_ref, np_ref, key_scr):
    b = pl.program_id(0)

    lane = jax.lax.broadcasted_iota(jnp.int32, (1, _PP), 1)
    real = lane < _P

    pr = priors_ref[...]
    al = jnp.transpose(armloc_ref[0])
    bc = pr[0:2] + al[0:2] * _VAR0 * pr[2:4]
    bwh = pr[2:4] * jnp.exp(al[2:4] * _VAR1)
    d1 = bc - bwh / 2.0
    d2 = bc + bwh / 2.0
    dx1, dy1 = d1[0:1], d1[1:2]
    dx2, dy2 = d2[0:1], d2[1:2]
    cs_c = (d2 + d1) / 2.0
    cs_wh = d2 - d1

    tg = targets_ref[0]
    tlabel = tg[:, 0:1]
    tx1 = tg[:, 1:2]
    ty1 = tg[:, 2:3]
    tx2 = tg[:, 3:4]
    ty2 = tg[:, 4:5]

    ones_row = jnp.ones((1, _PP), jnp.float32)

    def lanesplat(col):
        return jax.lax.dot_general(col, ones_row, (((1,), (0,)), ((), ())),
                                   preferred_element_type=jnp.float32)

    tx1b = lanesplat(tx1)
    ty1b = lanesplat(ty1)
    tx2b = lanesplat(tx2)
    ty2b = lanesplat(ty2)
    area_ab = lanesplat((tx2 - tx1) * (ty2 - ty1))

    iw = jnp.maximum(jnp.minimum(tx2b, dx2) - jnp.maximum(tx1b, dx1), 0.0)
    ih = jnp.maximum(jnp.minimum(ty2b, dy2) - jnp.maximum(ty1b, dy1), 0.0)
    inter = iw * ih
    area_b = (dx2 - dx1) * (dy2 - dy1)
    ov = inter / (area_ab + area_b - inter)

    t_iota = jax.lax.broadcasted_iota(jnp.int32, (_O, _PP), 0)
    big = jnp.int32(1 << 30)
    lane_f = lane.astype(jnp.float32)
    rowmax = jnp.max(ov, axis=1, keepdims=True)
    rowmask = ov == lanesplat(rowmax)
    bpi = jnp.min(jnp.where(rowmask, lane_f, jnp.float32(1 << 30)),
                  axis=1, keepdims=True)
    btv = jnp.max(ov, axis=0, keepdims=True)
    bti = jnp.min(jnp.where(ov == btv, t_iota, big), axis=0,
                  keepdims=True)
    claimed = lane_f == lanesplat(bpi)
    win_t = jnp.max(jnp.where(claimed, t_iota, -1), axis=0, keepdims=True)
    forced = win_t >= 0
    btv2 = jnp.where(forced, 2.0, btv)
    bti2 = jnp.where(forced, win_t, bti)

    onehot = (bti2 == t_iota).astype(jnp.float32)
    tgT = targetsT_ref[0]
    matched = jax.lax.dot_general(
        tgT, onehot, (((1,), (0,)), ((), ())),
        preferred_element_type=jnp.float32)
    mlab = matched[0:1]
    m1 = matched[1:3]
    m2 = matched[3:5]
    conf_t = jnp.where(btv2 < 0.5, 0, mlab.astype(jnp.int32) + 1)

    g_c = ((m1 + m2) / 2.0 - cs_c) / (_VAR0 * cs_wh)
    g_wh = jnp.log((m2 - m1) / cs_wh) / _VAR1

    ac = jnp.transpose(armconf_ref[0])
    am = jnp.maximum(ac[0:1], ac[1:2])
    e0 = jnp.exp(ac[0:1] - am)
    e1 = jnp.exp(ac[1:2] - am)
    p1 = e1 / (e0 + e1)
    pos = (conf_t > 0) & (p1 > _THETA)
    posf = pos.astype(jnp.float32)

    ol = jnp.transpose(odmloc_ref[0])

    def sl1(d):
        a = jnp.abs(d)
        return jnp.where(a < 1.0, 0.5 * a * a, a - 0.5)

    sboth = sl1(ol[0:2] - g_c) + sl1(ol[2:4] - g_wh)
    lloss = (sboth[0:1] + sboth[1:2]) * posf
    ll_row = jnp.sum(lloss, axis=1, keepdims=True)

    oc = jnp.transpose(odmconf_ref[0]).astype(jnp.float32)
    cm = jnp.max(oc, axis=0, keepdims=True)
    se = jnp.sum(jnp.exp(oc - cm), axis=0, keepdims=True)
    lse = jnp.log(se) + cm
    c_iota = jax.lax.broadcasted_iota(jnp.int32, (_C, _PP), 0)
    xt = jnp.sum(jnp.where(conf_t == c_iota, oc, 0.0), axis=0, keepdims=True)
    ce = jnp.where(real, lse - xt, 0.0)

    mining = jnp.maximum(jnp.where(pos, 0.0, ce), 0.0)
    mining = jnp.where(real, mining, 0.0)
    npos_i = jnp.sum(pos.astype(jnp.int32), axis=1, keepdims=True)
    k = jnp.minimum(_NEGPOS * npos_i, _P - 1)
    key = jax.lax.bitcast_convert_type(mining, jnp.int32)

    for i in range(8):
        key_scr[i:i + 1, :] = key[:, i * 2048:(i + 1) * 2048]
    key8 = key_scr[...]

    def bit_body(i, t):
        cand = t | (jnp.int32(1) << (jnp.int32(30) - i))
        cnt = jnp.sum((key8 >= cand).astype(jnp.int32), axis=(0, 1),
                      keepdims=True)
        return jnp.where(cnt >= k, cand, t)

    thr = jax.lax.fori_loop(0, 24, bit_body,
                            jnp.zeros((1, 1), jnp.int32))
    neg = key >= thr
    sel = jnp.maximum(posf, neg.astype(jnp.float32))
    lc_row = jnp.sum(ce * sel, axis=1, keepdims=True)
    np_row = jnp.sum(posf, axis=1, keepdims=True)

    @pl.when(b == 0)
    def _init():
        ll_ref[...] = jnp.zeros((1, 1), jnp.float32)
        lc_ref[...] = jnp.zeros((1, 1), jnp.float32)
        np_ref[...] = jnp.zeros((1, 1), jnp.float32)

    ll_ref[...] += ll_row
    lc_ref[...] += lc_row
    np_ref[...] += np_row


def _call(targets, targetsT, al, ac, olc, ocf, prs, interpret=False):
    return pl.pallas_call(
        _body,
        grid=(_B,),
        in_specs=[
            pl.BlockSpec((1, _O, 5), lambda b: (b, 0, 0)),
            pl.BlockSpec((1, 5, _O), lambda b: (b, 0, 0)),
            pl.BlockSpec((1, _PP, 4), lambda b: (b, 0, 0)),
            pl.BlockSpec((1, _PP, 2), lambda b: (b, 0, 0)),
            pl.BlockSpec((1, _PP, 4), lambda b: (b, 0, 0)),
            pl.BlockSpec((1, _PP, _C), lambda b: (b, 0, 0)),
            pl.BlockSpec((4, _PP), lambda b: (0, 0)),
        ],
        out_specs=[
            pl.BlockSpec((1, 1), lambda b: (0, 0)),
            pl.BlockSpec((1, 1), lambda b: (0, 0)),
            pl.BlockSpec((1, 1), lambda b: (0, 0)),
        ],
        out_shape=[
            jax.ShapeDtypeStruct((1, 1), jnp.float32),
            jax.ShapeDtypeStruct((1, 1), jnp.float32),
            jax.ShapeDtypeStruct((1, 1), jnp.float32),
        ],
        scratch_shapes=[pltpu.VMEM((8, _PP // 8), jnp.int32)],
        compiler_params=pltpu.CompilerParams(
            vmem_limit_bytes=100 * 1024 * 1024),
        interpret=interpret,
    )(targets, targetsT, al, ac, olc, ocf, prs)


def _run(arm_loc_data, arm_conf_data, odm_loc_data, odm_conf_data, priors,
         targets, interpret=False):
    pad = _PP - _P
    loc_pad = jnp.broadcast_to(
        jnp.array([-1000.0, -1000.0, 0.0, 0.0], jnp.float32), (_B, pad, 4))
    al = jnp.concatenate([arm_loc_data, loc_pad], axis=1)
    ac = jnp.concatenate(
        [arm_conf_data, jnp.zeros((_B, pad, 2), jnp.float32)], axis=1)
    olc = jnp.concatenate(
        [odm_loc_data, jnp.zeros((_B, pad, 4), jnp.float32)], axis=1)
    ocf = jnp.concatenate(
        [odm_conf_data.astype(jnp.bfloat16),
         jnp.zeros((_B, pad, _C), jnp.bfloat16)], axis=1)
    prs = jnp.concatenate(
        [priors,
         jnp.broadcast_to(jnp.array([0.0, 0.0, 1.0, 1.0], jnp.float32),
                          (pad, 4))], axis=0).T

    targetsT = targets.transpose(0, 2, 1)
    ll, lc, npos = _call(targets, targetsT, al, ac, olc, ocf, prs,
                         interpret=interpret)
    n = npos[0, 0]
    return ll[0, 0] / n, lc[0, 0] / n


def kernel(arm_loc_data, arm_conf_data, odm_loc_data, odm_conf_data, priors,
           targets):
    return _run(arm_loc_data, arm_conf_data, odm_loc_data, odm_conf_data,
                priors, targets)


# speedup vs baseline: 5.1420x; 5.1420x over previous
import jax
import jax.numpy as jnp
from jax.experimental import pallas as pl
from jax.experimental.pallas import tpu as pltpu

_P = 16320
_PP = 16384
_B = 32
_O = 50
_C = 21
_VAR0, _VAR1 = 0.1, 0.2
_THETA = 0.01
_NEGPOS = 3


def _body(targets_ref, targetsT_ref, armloc_ref, armconf_ref, odmloc_ref,
          odmconf_ref, priors_ref, ll_ref, lc_ref, np_ref, key_scr):
    b = pl.program_id(0)

    lane = jax.lax.broadcasted_iota(jnp.int32, (1, _PP), 1)
    real = lane < _P

    pr = priors_ref[...]
    al = armloc_ref[0]
    bc = pr[0:2] + al[0:2] * _VAR0 * pr[2:4]
    bwh = pr[2:4] * jnp.exp(al[2:4] * _VAR1)
    d1 = bc - bwh / 2.0
    d2 = bc + bwh / 2.0
    dx1, dy1 = d1[0:1], d1[1:2]
    dx2, dy2 = d2[0:1], d2[1:2]
    cs_c = (d2 + d1) / 2.0
    cs_wh = d2 - d1

    tg = targets_ref[0]
    tlabel = tg[:, 0:1]
    tx1 = tg[:, 1:2]
    ty1 = tg[:, 2:3]
    tx2 = tg[:, 3:4]
    ty2 = tg[:, 4:5]

    ones_row = jnp.ones((1, _PP), jnp.float32)

    def lanesplat(col):
        return jax.lax.dot_general(col, ones_row, (((1,), (0,)), ((), ())),
                                   preferred_element_type=jnp.float32)

    tx1b = lanesplat(tx1)
    ty1b = lanesplat(ty1)
    tx2b = lanesplat(tx2)
    ty2b = lanesplat(ty2)
    area_ab = lanesplat((tx2 - tx1) * (ty2 - ty1))

    iw = jnp.maximum(jnp.minimum(tx2b, dx2) - jnp.maximum(tx1b, dx1), 0.0)
    ih = jnp.maximum(jnp.minimum(ty2b, dy2) - jnp.maximum(ty1b, dy1), 0.0)
    inter = iw * ih
    area_b = (dx2 - dx1) * (dy2 - dy1)
    ov = inter / (area_ab + area_b - inter)

    t_iota = jax.lax.broadcasted_iota(jnp.int32, (_O, _PP), 0)
    big = jnp.int32(1 << 30)
    lane_f = lane.astype(jnp.float32)
    rowmax = jnp.max(ov, axis=1, keepdims=True)
    rowmask = ov == lanesplat(rowmax)
    bpi = jnp.min(jnp.where(rowmask, lane_f, jnp.float32(1 << 30)),
                  axis=1, keepdims=True)
    btv = jnp.max(ov, axis=0, keepdims=True)
    bti = jnp.min(jnp.where(ov == btv, t_iota, big), axis=0,
                  keepdims=True)
    claimed = lane_f == lanesplat(bpi)
    win_t = jnp.max(jnp.where(claimed, t_iota, -1), axis=0, keepdims=True)
    forced = win_t >= 0
    btv2 = jnp.where(forced, 2.0, btv)
    bti2 = jnp.where(forced, win_t, bti)

    onehot = (bti2 == t_iota).astype(jnp.float32)
    tgT = targetsT_ref[0]
    matched = jax.lax.dot_general(
        tgT, onehot, (((1,), (0,)), ((), ())),
        preferred_element_type=jnp.float32)
    mlab = matched[0:1]
    m1 = matched[1:3]
    m2 = matched[3:5]
    conf_t = jnp.where(btv2 < 0.5, 0, mlab.astype(jnp.int32) + 1)

    g_c = ((m1 + m2) / 2.0 - cs_c) / (_VAR0 * cs_wh)
    g_wh = jnp.log((m2 - m1) / cs_wh) / _VAR1

    ac = armconf_ref[0]
    am = jnp.maximum(ac[0:1], ac[1:2])
    e0 = jnp.exp(ac[0:1] - am)
    e1 = jnp.exp(ac[1:2] - am)
    p1 = e1 / (e0 + e1)
    pos = (conf_t > 0) & (p1 > _THETA)
    posf = pos.astype(jnp.float32)

    ol = odmloc_ref[0]

    def sl1(d):
        a = jnp.abs(d)
        return jnp.where(a < 1.0, 0.5 * a * a, a - 0.5)

    sboth = sl1(ol[0:2] - g_c) + sl1(ol[2:4] - g_wh)
    lloss = (sboth[0:1] + sboth[1:2]) * posf
    ll_row = jnp.sum(lloss, axis=1, keepdims=True)

    oc = odmconf_ref[0].astype(jnp.float32)
    cm = jnp.max(oc, axis=0, keepdims=True)
    se = jnp.sum(jnp.exp(oc - cm), axis=0, keepdims=True)
    lse = jnp.log(se) + cm
    c_iota = jax.lax.broadcasted_iota(jnp.int32, (_C, _PP), 0)
    xt = jnp.sum(jnp.where(conf_t == c_iota, oc, 0.0), axis=0, keepdims=True)
    ce = jnp.where(real, lse - xt, 0.0)

    mining = jnp.maximum(jnp.where(pos, 0.0, ce), 0.0)
    mining = jnp.where(real, mining, 0.0)
    npos_i = jnp.sum(pos.astype(jnp.int32), axis=1, keepdims=True)
    k = jnp.minimum(_NEGPOS * npos_i, _P - 1)
    key = jax.lax.bitcast_convert_type(mining, jnp.int32)

    for i in range(8):
        key_scr[i:i + 1, :] = key[:, i * 2048:(i + 1) * 2048]
    key8 = key_scr[...]

    def bit_body(i, t):
        cand = t | (jnp.int32(1) << (jnp.int32(30) - i))
        cnt = jnp.sum((key8 >= cand).astype(jnp.int32), axis=(0, 1),
                      keepdims=True)
        return jnp.where(cnt >= k, cand, t)

    thr = jax.lax.fori_loop(0, 24, bit_body,
                            jnp.zeros((1, 1), jnp.int32))
    neg = key >= thr
    sel = jnp.maximum(posf, neg.astype(jnp.float32))
    lc_row = jnp.sum(ce * sel, axis=1, keepdims=True)
    np_row = jnp.sum(posf, axis=1, keepdims=True)

    @pl.when(b == 0)
    def _init():
        ll_ref[...] = jnp.zeros((1, 1), jnp.float32)
        lc_ref[...] = jnp.zeros((1, 1), jnp.float32)
        np_ref[...] = jnp.zeros((1, 1), jnp.float32)

    ll_ref[...] += ll_row
    lc_ref[...] += lc_row
    np_ref[...] += np_row


def _call(targets, targetsT, al, ac, olc, ocf, prs, interpret=False):
    return pl.pallas_call(
        _body,
        grid=(_B,),
        in_specs=[
            pl.BlockSpec((1, _O, 5), lambda b: (b, 0, 0)),
            pl.BlockSpec((1, 5, _O), lambda b: (b, 0, 0)),
            pl.BlockSpec((1, 4, _PP), lambda b: (b, 0, 0)),
            pl.BlockSpec((1, 2, _PP), lambda b: (b, 0, 0)),
            pl.BlockSpec((1, 4, _PP), lambda b: (b, 0, 0)),
            pl.BlockSpec((1, _C, _PP), lambda b: (b, 0, 0)),
            pl.BlockSpec((4, _PP), lambda b: (0, 0)),
        ],
        out_specs=[
            pl.BlockSpec((1, 1), lambda b: (0, 0)),
            pl.BlockSpec((1, 1), lambda b: (0, 0)),
            pl.BlockSpec((1, 1), lambda b: (0, 0)),
        ],
        out_shape=[
            jax.ShapeDtypeStruct((1, 1), jnp.float32),
            jax.ShapeDtypeStruct((1, 1), jnp.float32),
            jax.ShapeDtypeStruct((1, 1), jnp.float32),
        ],
        scratch_shapes=[pltpu.VMEM((8, _PP // 8), jnp.int32)],
        compiler_params=pltpu.CompilerParams(
            vmem_limit_bytes=100 * 1024 * 1024),
        interpret=interpret,
    )(targets, targetsT, al, ac, olc, ocf, prs)


def _run(arm_loc_data, arm_conf_data, odm_loc_data, odm_conf_data, priors,
         targets, interpret=False):
    pad = _PP - _P
    loc_pad = jnp.broadcast_to(
        jnp.array([-1000.0, -1000.0, 0.0, 0.0], jnp.float32), (_B, pad, 4))
    al = jnp.concatenate([arm_loc_data, loc_pad], axis=1).transpose(0, 2, 1)
    ac = jnp.concatenate(
        [arm_conf_data, jnp.zeros((_B, pad, 2), jnp.float32)],
        axis=1).transpose(0, 2, 1)
    olc = jnp.concatenate(
        [odm_loc_data, jnp.zeros((_B, pad, 4), jnp.float32)],
        axis=1).transpose(0, 2, 1)
    ocf = jnp.concatenate(
        [odm_conf_data.astype(jnp.bfloat16),
         jnp.zeros((_B, pad, _C), jnp.bfloat16)],
        axis=1).transpose(0, 2, 1)
    prs = jnp.concatenate(
        [priors,
         jnp.broadcast_to(jnp.array([0.0, 0.0, 1.0, 1.0], jnp.float32),
                          (pad, 4))], axis=0).T

    targetsT = targets.transpose(0, 2, 1)
    ll, lc, npos = _call(targets, targetsT, al, ac, olc, ocf, prs,
                         interpret=interpret)
    n = npos[0, 0]
    return ll[0, 0] / n, lc[0, 0] / n


def kernel(arm_loc_data, arm_conf_data, odm_loc_data, odm_conf_data, priors,
           targets):
    return _run(arm_loc_data, arm_conf_data, odm_loc_data, odm_conf_data,
                priors, targets)


# speedup vs baseline: 5.5380x; 1.0770x over previous
import jax
import jax.numpy as jnp
from jax.experimental import pallas as pl
from jax.experimental.pallas import tpu as pltpu

_P = 16320
_PP = 16384
_B = 32
_O = 50
_C = 21
_VAR0, _VAR1 = 0.1, 0.2
_THETA = 0.01
_NEGPOS = 3


def _body(targets_ref, targetsT_ref, armloc_ref, armconf_ref, odmloc_ref,
          odmconf_ref, priors_ref, ll_ref, lc_ref, np_ref, key_scr):
    b = pl.program_id(0)

    lane = jax.lax.broadcasted_iota(jnp.int32, (1, _P), 1)

    pr = priors_ref[...]
    al = armloc_ref[0]
    bc = pr[0:2] + al[0:2] * _VAR0 * pr[2:4]
    bwh = pr[2:4] * jnp.exp(al[2:4] * _VAR1)
    d1 = bc - bwh / 2.0
    d2 = bc + bwh / 2.0
    dx1, dy1 = d1[0:1], d1[1:2]
    dx2, dy2 = d2[0:1], d2[1:2]
    cs_c = (d2 + d1) / 2.0
    cs_wh = d2 - d1

    tg = targets_ref[0]
    tlabel = tg[:, 0:1]
    tx1 = tg[:, 1:2]
    ty1 = tg[:, 2:3]
    tx2 = tg[:, 3:4]
    ty2 = tg[:, 4:5]

    ones_row = jnp.ones((1, _P), jnp.float32)

    def lanesplat(col):
        return jax.lax.dot_general(col, ones_row, (((1,), (0,)), ((), ())),
                                   preferred_element_type=jnp.float32)

    tx1b = lanesplat(tx1)
    ty1b = lanesplat(ty1)
    tx2b = lanesplat(tx2)
    ty2b = lanesplat(ty2)
    area_ab = lanesplat((tx2 - tx1) * (ty2 - ty1))

    iw = jnp.maximum(jnp.minimum(tx2b, dx2) - jnp.maximum(tx1b, dx1), 0.0)
    ih = jnp.maximum(jnp.minimum(ty2b, dy2) - jnp.maximum(ty1b, dy1), 0.0)
    inter = iw * ih
    area_b = (dx2 - dx1) * (dy2 - dy1)
    ov = inter / (area_ab + area_b - inter)

    t_iota = jax.lax.broadcasted_iota(jnp.int32, (_O, _P), 0)
    big = jnp.int32(1 << 30)
    lane_f = lane.astype(jnp.float32)
    rowmax = jnp.max(ov, axis=1, keepdims=True)
    rowmask = ov == lanesplat(rowmax)
    bpi = jnp.min(jnp.where(rowmask, lane_f, jnp.float32(1 << 30)),
                  axis=1, keepdims=True)
    btv = jnp.max(ov, axis=0, keepdims=True)
    bti = jnp.min(jnp.where(ov == btv, t_iota, big), axis=0,
                  keepdims=True)
    claimed = lane_f == lanesplat(bpi)
    win_t = jnp.max(jnp.where(claimed, t_iota, -1), axis=0, keepdims=True)
    forced = win_t >= 0
    btv2 = jnp.where(forced, 2.0, btv)
    bti2 = jnp.where(forced, win_t, bti)

    onehot = (bti2 == t_iota).astype(jnp.float32)
    tgT = targetsT_ref[0]
    matched = jax.lax.dot_general(
        tgT, onehot, (((1,), (0,)), ((), ())),
        preferred_element_type=jnp.float32)
    mlab = matched[0:1]
    m1 = matched[1:3]
    m2 = matched[3:5]
    conf_t = jnp.where(btv2 < 0.5, 0, mlab.astype(jnp.int32) + 1)

    g_c = ((m1 + m2) / 2.0 - cs_c) / (_VAR0 * cs_wh)
    g_wh = jnp.log((m2 - m1) / cs_wh) / _VAR1

    ac = armconf_ref[0]
    am = jnp.maximum(ac[0:1], ac[1:2])
    e0 = jnp.exp(ac[0:1] - am)
    e1 = jnp.exp(ac[1:2] - am)
    p1 = e1 / (e0 + e1)
    pos = (conf_t > 0) & (p1 > _THETA)
    posf = pos.astype(jnp.float32)

    ol = odmloc_ref[0]

    def sl1(d):
        a = jnp.abs(d)
        return jnp.where(a < 1.0, 0.5 * a * a, a - 0.5)

    sboth = sl1(ol[0:2] - g_c) + sl1(ol[2:4] - g_wh)
    lloss = (sboth[0:1] + sboth[1:2]) * posf
    ll_row = jnp.sum(lloss, axis=1, keepdims=True)

    oc = odmconf_ref[0].astype(jnp.float32)
    cm = jnp.max(oc, axis=0, keepdims=True)
    se = jnp.sum(jnp.exp(oc - cm), axis=0, keepdims=True)
    lse = jnp.log(se) + cm
    c_iota = jax.lax.broadcasted_iota(jnp.int32, (_C, _P), 0)
    xt = jnp.sum(jnp.where(conf_t == c_iota, oc, 0.0), axis=0, keepdims=True)
    ce = lse - xt

    mining = jnp.maximum(jnp.where(pos, 0.0, ce), 0.0)
    npos_i = jnp.sum(pos.astype(jnp.int32), axis=1, keepdims=True)
    k = jnp.minimum(_NEGPOS * npos_i, _P - 1)
    key = jax.lax.bitcast_convert_type(mining, jnp.int32)

    for i in range(7):
        key_scr[i:i + 1, :] = key[:, i * 2048:(i + 1) * 2048]
    key_scr[7:8, :] = jnp.zeros((1, 2048), jnp.int32)
    key_scr[7:8, 0:_P - 7 * 2048] = key[:, 7 * 2048:_P]
    key8 = key_scr[...]

    def bit_body(i, t):
        cand = t | (jnp.int32(1) << (jnp.int32(30) - i))
        cnt = jnp.sum((key8 >= cand).astype(jnp.int32), axis=(0, 1),
                      keepdims=True)
        return jnp.where(cnt >= k, cand, t)

    thr = jax.lax.fori_loop(0, 24, bit_body,
                            jnp.zeros((1, 1), jnp.int32))
    neg = key >= thr
    sel = jnp.maximum(posf, neg.astype(jnp.float32))
    lc_row = jnp.sum(ce * sel, axis=1, keepdims=True)
    np_row = jnp.sum(posf, axis=1, keepdims=True)

    @pl.when(b == 0)
    def _init():
        ll_ref[...] = jnp.zeros((1, 1), jnp.float32)
        lc_ref[...] = jnp.zeros((1, 1), jnp.float32)
        np_ref[...] = jnp.zeros((1, 1), jnp.float32)

    ll_ref[...] += ll_row
    lc_ref[...] += lc_row
    np_ref[...] += np_row


def _call(targets, targetsT, al, ac, olc, ocf, prs, interpret=False):
    return pl.pallas_call(
        _body,
        grid=(_B,),
        in_specs=[
            pl.BlockSpec((1, _O, 5), lambda b: (b, 0, 0)),
            pl.BlockSpec((1, 5, _O), lambda b: (b, 0, 0)),
            pl.BlockSpec((1, 4, _P), lambda b: (b, 0, 0)),
            pl.BlockSpec((1, 2, _P), lambda b: (b, 0, 0)),
            pl.BlockSpec((1, 4, _P), lambda b: (b, 0, 0)),
            pl.BlockSpec((1, _C, _P), lambda b: (b, 0, 0)),
            pl.BlockSpec((4, _P), lambda b: (0, 0)),
        ],
        out_specs=[
            pl.BlockSpec((1, 1), lambda b: (0, 0)),
            pl.BlockSpec((1, 1), lambda b: (0, 0)),
            pl.BlockSpec((1, 1), lambda b: (0, 0)),
        ],
        out_shape=[
            jax.ShapeDtypeStruct((1, 1), jnp.float32),
            jax.ShapeDtypeStruct((1, 1), jnp.float32),
            jax.ShapeDtypeStruct((1, 1), jnp.float32),
        ],
        scratch_shapes=[pltpu.VMEM((8, 2048), jnp.int32)],
        compiler_params=pltpu.CompilerParams(
            vmem_limit_bytes=100 * 1024 * 1024),
        interpret=interpret,
    )(targets, targetsT, al, ac, olc, ocf, prs)


def _run(arm_loc_data, arm_conf_data, odm_loc_data, odm_conf_data, priors,
         targets, interpret=False):
    al = arm_loc_data.transpose(0, 2, 1)
    ac = arm_conf_data.transpose(0, 2, 1)
    olc = odm_loc_data.transpose(0, 2, 1)
    ocf = odm_conf_data.astype(jnp.bfloat16).transpose(0, 2, 1)
    prs = priors.T

    targetsT = targets.transpose(0, 2, 1)
    ll, lc, npos = _call(targets, targetsT, al, ac, olc, ocf, prs,
                         interpret=interpret)
    n = npos[0, 0]
    return ll[0, 0] / n, lc[0, 0] / n


def kernel(arm_loc_data, arm_conf_data, odm_loc_data, odm_conf_data, priors,
           targets):
    return _run(arm_loc_data, arm_conf_data, odm_loc_data, odm_conf_data,
                priors, targets)
